# Initial kernel scaffold; baseline (speedup 1.0000x reference)
import jax
import jax.numpy as jnp
from jax.experimental import pallas as pl

NFEAT = 256
NHID = 32
NCLASS = 64
NHEADS = 8
ALPHA = 0.2
B = 4
N = 512


def _masked_att(h, a_src_row, a_dst_row, mask):
    f = jnp.sum(h * a_src_row[None, :], axis=1, keepdims=True)
    g = jnp.sum(h * a_dst_row[None, :], axis=1, keepdims=True)
    z = f + g.T
    e = jnp.exp(-jnp.where(z >= 0, z, ALPHA * z)) * mask
    rowsum = jnp.sum(e, axis=1, keepdims=True)
    hp = jnp.dot(e, h, preferred_element_type=jnp.float32)
    return hp / rowsum


def _gat_body(x_ref, adj_ref, wc_ref, aatt_ref, wout_ref, aout_ref, out_ref):
    xb = x_ref[0]
    mask = (adj_ref[...] != 0).astype(jnp.float32)

    h_all = jnp.dot(xb, wc_ref[...], preferred_element_type=jnp.float32)

    head_outs = []
    for hi in range(NHEADS):
        h = h_all[:, hi * NHID:(hi + 1) * NHID]
        a_src = aatt_ref[hi, :NHID]
        a_dst = aatt_ref[hi, NHID:]
        head_outs.append(jax.nn.elu(_masked_att(h, a_src, a_dst, mask)))
    x1 = jnp.concatenate(head_outs, axis=1)

    h2 = jnp.dot(x1, wout_ref[...], preferred_element_type=jnp.float32)
    a2_src = aout_ref[0, :NCLASS]
    a2_dst = aout_ref[0, NCLASS:]
    out_ref[0] = jax.nn.elu(_masked_att(h2, a2_src, a2_dst, mask))


def kernel(x, adj, W_att, a_att, W_out, a_out):
    wc = W_att.transpose(1, 0, 2).reshape(NFEAT, NHEADS * NHID)
    aatt = a_att[:, 0, :]
    aout = jnp.concatenate(
        [a_out, jnp.zeros((7, 2 * NCLASS), dtype=a_out.dtype)], axis=0)

    grid = (B,)
    return pl.pallas_call(
        _gat_body,
        grid=grid,
        in_specs=[
            pl.BlockSpec((1, N, NFEAT), lambda b: (b, 0, 0)),
            pl.BlockSpec((N, N), lambda b: (0, 0)),
            pl.BlockSpec((NFEAT, NHEADS * NHID), lambda b: (0, 0)),
            pl.BlockSpec((NHEADS, 2 * NHID), lambda b: (0, 0)),
            pl.BlockSpec((NHEADS * NHID, NCLASS), lambda b: (0, 0)),
            pl.BlockSpec((NHEADS, 2 * NCLASS), lambda b: (0, 0)),
        ],
        out_specs=pl.BlockSpec((1, N, NCLASS), lambda b: (b, 0, 0)),
        out_shape=jax.ShapeDtypeStruct((B, N, NCLASS), jnp.float32),
    )(x, adj, wc, aatt, W_out, aout)


# speedup vs baseline: 1528.9643x; 1528.9643x over previous
import jax
import jax.numpy as jnp
from jax.experimental import pallas as pl

NFEAT = 256
NHID = 32
NCLASS = 64
NHEADS = 8
ALPHA = 0.2
B = 4
N = 512


def _elu(v):
    return jnp.where(v > 0, v, jnp.exp(jnp.minimum(v, 0.0)) - 1.0)


def _masked_att(h, a_src_row, a_dst_row, mask):
    f = jnp.sum(h * a_src_row[None, :], axis=1, keepdims=True)
    g = jnp.sum(h * a_dst_row[None, :], axis=1, keepdims=True)
    z = f + g.T
    e = jnp.exp(-jnp.where(z >= 0, z, ALPHA * z)) * mask
    rowsum = jnp.sum(e, axis=1, keepdims=True)
    hp = jnp.dot(e, h, preferred_element_type=jnp.float32)
    return hp / rowsum


def _gat_body(x_ref, adj_ref, wc_ref, aatt_ref, wout_ref, aout_ref, out_ref):
    xb = x_ref[0]
    mask = (adj_ref[...] != 0).astype(jnp.float32)

    h_all = jnp.dot(xb, wc_ref[...], preferred_element_type=jnp.float32)

    head_outs = []
    for hi in range(NHEADS):
        h = h_all[:, hi * NHID:(hi + 1) * NHID]
        a_src = aatt_ref[hi, :NHID]
        a_dst = aatt_ref[hi, NHID:]
        head_outs.append(_elu(_masked_att(h, a_src, a_dst, mask)))
    x1 = jnp.concatenate(head_outs, axis=1)

    h2 = jnp.dot(x1, wout_ref[...], preferred_element_type=jnp.float32)
    a2_src = aout_ref[0, :NCLASS]
    a2_dst = aout_ref[0, NCLASS:]
    out_ref[0] = _elu(_masked_att(h2, a2_src, a2_dst, mask))


def kernel(x, adj, W_att, a_att, W_out, a_out):
    wc = W_att.transpose(1, 0, 2).reshape(NFEAT, NHEADS * NHID)
    aatt = a_att[:, 0, :]
    aout = jnp.concatenate(
        [a_out, jnp.zeros((7, 2 * NCLASS), dtype=a_out.dtype)], axis=0)

    grid = (B,)
    return pl.pallas_call(
        _gat_body,
        grid=grid,
        in_specs=[
            pl.BlockSpec((1, N, NFEAT), lambda b: (b, 0, 0)),
            pl.BlockSpec((N, N), lambda b: (0, 0)),
            pl.BlockSpec((NFEAT, NHEADS * NHID), lambda b: (0, 0)),
            pl.BlockSpec((NHEADS, 2 * NHID), lambda b: (0, 0)),
            pl.BlockSpec((NHEADS * NHID, NCLASS), lambda b: (0, 0)),
            pl.BlockSpec((NHEADS, 2 * NCLASS), lambda b: (0, 0)),
        ],
        out_specs=pl.BlockSpec((1, N, NCLASS), lambda b: (b, 0, 0)),
        out_shape=jax.ShapeDtypeStruct((B, N, NCLASS), jnp.float32),
    )(x, adj, wc, aatt, W_out, aout)


# speedup vs baseline: 1935.3720x; 1.2658x over previous
import jax
import jax.numpy as jnp
from jax.experimental import pallas as pl

NFEAT = 256
NHID = 32
NCLASS = 64
NHEADS = 8
ALPHA = 0.2
B = 4
N = 512


def _elu(v):
    return jnp.where(v > 0, v, jnp.exp(jnp.minimum(v, 0.0)) - 1.0)


def _att_prop(f, gT, h, mask, d):
    p = jnp.exp(-f)
    pa = jnp.exp(-ALPHA * f)
    q = jnp.exp(-gT)
    qa = jnp.exp(-ALPHA * gT)
    cond = f >= -gT
    e = jnp.where(cond, p * q, pa * qa) * mask
    lane = jax.lax.broadcasted_iota(jnp.int32, (N, d), 1)
    ones_col = (lane == 0).astype(jnp.float32)
    rhs = jnp.concatenate([h, ones_col], axis=1)
    acc = jnp.dot(e.astype(jnp.bfloat16), rhs.astype(jnp.bfloat16),
                  preferred_element_type=jnp.float32)
    hp = acc[:, :d]
    rowsum = acc[:, d:d + 1]
    return hp / rowsum


def _gat_body(x_ref, adj_ref, wc_ref, acomb_ref, wout_ref, a2_ref, out_ref):
    xb = x_ref[0]
    mask = (adj_ref[...] != 0).astype(jnp.float32)

    h_all = jnp.dot(xb, wc_ref[...], preferred_element_type=jnp.float32)
    fg = jnp.dot(h_all, acomb_ref[...],
                 preferred_element_type=jnp.float32)
    fgT = fg.T

    head_outs = []
    for hi in range(NHEADS):
        h = h_all[:, hi * NHID:(hi + 1) * NHID]
        f = fg[:, hi:hi + 1]
        gT = fgT[NHEADS + hi:NHEADS + hi + 1, :]
        head_outs.append(_elu(_att_prop(f, gT, h, mask, NHID)))
    x1 = jnp.concatenate(head_outs, axis=1)

    h2 = jnp.dot(x1, wout_ref[...], preferred_element_type=jnp.float32)
    fg2 = jnp.dot(h2, a2_ref[...],
                  preferred_element_type=jnp.float32)
    fg2T = fg2.T
    f2 = fg2[:, 0:1]
    g2T = fg2T[1:2, :]
    out_ref[0] = _elu(_att_prop(f2, g2T, h2, mask, NCLASS))


def kernel(x, adj, W_att, a_att, W_out, a_out):
    wc = W_att.transpose(1, 0, 2).reshape(NFEAT, NHEADS * NHID)
    a_src = a_att[:, 0, :NHID]
    a_dst = a_att[:, 0, NHID:]
    eye = jnp.eye(NHEADS, dtype=a_att.dtype)
    acomb = jnp.concatenate(
        [(a_src[:, :, None] * eye[:, None, :]).reshape(NHEADS * NHID, NHEADS),
         (a_dst[:, :, None] * eye[:, None, :]).reshape(NHEADS * NHID, NHEADS)],
        axis=1)
    a2 = jnp.concatenate(
        [a_out[0, :NCLASS, None], a_out[0, NCLASS:, None],
         jnp.zeros((NCLASS, 6), dtype=a_out.dtype)], axis=1)

    grid = (B,)
    return pl.pallas_call(
        _gat_body,
        grid=grid,
        in_specs=[
            pl.BlockSpec((1, N, NFEAT), lambda b: (b, 0, 0)),
            pl.BlockSpec((N, N), lambda b: (0, 0)),
            pl.BlockSpec((NFEAT, NHEADS * NHID), lambda b: (0, 0)),
            pl.BlockSpec((NHEADS * NHID, 2 * NHEADS), lambda b: (0, 0)),
            pl.BlockSpec((NHEADS * NHID, NCLASS), lambda b: (0, 0)),
            pl.BlockSpec((NCLASS, 8), lambda b: (0, 0)),
        ],
        out_specs=pl.BlockSpec((1, N, NCLASS), lambda b: (b, 0, 0)),
        out_shape=jax.ShapeDtypeStruct((B, N, NCLASS), jnp.float32),
    )(x, adj, wc, acomb, W_out, a2)


# speedup vs baseline: 2216.6503x; 1.1453x over previous
import jax
import jax.numpy as jnp
from jax.experimental import pallas as pl

NFEAT = 256
NHID = 32
NCLASS = 64
NHEADS = 8
ALPHA = 0.2
B = 4
N = 512


def _elu(v):
    return jnp.where(v > 0, v, jnp.exp(jnp.minimum(v, 0.0)) - 1.0)


def _att_prop(p, pa, q, qa, h, mask, d):
    pq = p * q
    e = jnp.where(pq <= 1.0, pq, pa * qa) * mask
    lane = jax.lax.broadcasted_iota(jnp.int32, (N, d), 1)
    ones_col = (lane == 0).astype(jnp.float32)
    rhs = jnp.concatenate([h, ones_col], axis=1)
    acc = jnp.dot(e.astype(jnp.bfloat16), rhs.astype(jnp.bfloat16),
                  preferred_element_type=jnp.float32)
    hp = acc[:, :d]
    rowsum = acc[:, d:d + 1]
    return hp * (1.0 / rowsum)


def _gat_body(x_ref, adj_ref, wc_ref, acomb_ref, wout_ref, a2_ref, out_ref):
    xb = x_ref[0]
    mask = (adj_ref[...] != 0).astype(jnp.float32)

    h_all = jnp.dot(xb, wc_ref[...], preferred_element_type=jnp.float32)
    fg = jnp.dot(h_all, acomb_ref[...],
                 preferred_element_type=jnp.float32)
    fgT = fg.T
    P = jnp.exp(-fg)
    PA = jnp.exp(-ALPHA * fg)
    Q = jnp.exp(-fgT)
    QA = jnp.exp(-ALPHA * fgT)

    head_outs = []
    for hi in range(NHEADS):
        h = h_all[:, hi * NHID:(hi + 1) * NHID]
        p = P[:, hi:hi + 1]
        pa = PA[:, hi:hi + 1]
        q = Q[NHEADS + hi:NHEADS + hi + 1, :]
        qa = QA[NHEADS + hi:NHEADS + hi + 1, :]
        head_outs.append(_elu(_att_prop(p, pa, q, qa, h, mask, NHID)))
    x1 = jnp.concatenate(head_outs, axis=1)

    h2 = jnp.dot(x1, wout_ref[...], preferred_element_type=jnp.float32)
    fg2 = jnp.dot(h2, a2_ref[...],
                  preferred_element_type=jnp.float32)
    fg2T = fg2.T
    f2 = fg2[:, 0:1]
    g2T = fg2T[1:2, :]
    out_ref[0] = _elu(_att_prop(
        jnp.exp(-f2), jnp.exp(-ALPHA * f2),
        jnp.exp(-g2T), jnp.exp(-ALPHA * g2T), h2, mask, NCLASS))


def kernel(x, adj, W_att, a_att, W_out, a_out):
    wc = W_att.transpose(1, 0, 2).reshape(NFEAT, NHEADS * NHID)
    a_src = a_att[:, 0, :NHID]
    a_dst = a_att[:, 0, NHID:]
    eye = jnp.eye(NHEADS, dtype=a_att.dtype)
    acomb = jnp.concatenate(
        [(a_src[:, :, None] * eye[:, None, :]).reshape(NHEADS * NHID, NHEADS),
         (a_dst[:, :, None] * eye[:, None, :]).reshape(NHEADS * NHID, NHEADS)],
        axis=1)
    a2 = jnp.concatenate(
        [a_out[0, :NCLASS, None], a_out[0, NCLASS:, None],
         jnp.zeros((NCLASS, 6), dtype=a_out.dtype)], axis=1)

    grid = (B,)
    return pl.pallas_call(
        _gat_body,
        grid=grid,
        in_specs=[
            pl.BlockSpec((1, N, NFEAT), lambda b: (b, 0, 0)),
            pl.BlockSpec((N, N), lambda b: (0, 0)),
            pl.BlockSpec((NFEAT, NHEADS * NHID), lambda b: (0, 0)),
            pl.BlockSpec((NHEADS * NHID, 2 * NHEADS), lambda b: (0, 0)),
            pl.BlockSpec((NHEADS * NHID, NCLASS), lambda b: (0, 0)),
            pl.BlockSpec((NCLASS, 8), lambda b: (0, 0)),
        ],
        out_specs=pl.BlockSpec((1, N, NCLASS), lambda b: (b, 0, 0)),
        out_shape=jax.ShapeDtypeStruct((B, N, NCLASS), jnp.float32),
    )(x, adj, wc, acomb, W_out, a2)


# speedup vs baseline: 2227.3723x; 1.0048x over previous
import jax
import jax.numpy as jnp
from jax.experimental import pallas as pl

NFEAT = 256
NHID = 32
NCLASS = 64
NHEADS = 8
ALPHA = 0.2
B = 4
N = 512


def _elu(v):
    return jnp.where(v > 0, v, jnp.exp(jnp.minimum(v, 0.0)) - 1.0)


def _att_prop(p, pa, q, qa, h, mask, d):
    e = jnp.minimum(p * q, pa * qa) * mask
    lane = jax.lax.broadcasted_iota(jnp.int32, (N, d), 1)
    ones_col = (lane == 0).astype(jnp.float32)
    rhs = jnp.concatenate([h, ones_col], axis=1)
    acc = jnp.dot(e.astype(jnp.bfloat16), rhs.astype(jnp.bfloat16),
                  preferred_element_type=jnp.float32)
    hp = acc[:, :d]
    rowsum = acc[:, d:d + 1]
    return hp * (1.0 / rowsum)


def _gat_body(x_ref, adj_ref, wc_ref, acomb_ref, wout_ref, a2_ref, out_ref):
    xb = x_ref[0]
    mask = (adj_ref[...] != 0).astype(jnp.float32)

    h_all = jnp.dot(xb, wc_ref[...], preferred_element_type=jnp.float32)
    fg = jnp.dot(h_all, acomb_ref[...],
                 preferred_element_type=jnp.float32)
    fgT = fg.T
    P = jnp.exp(-fg)
    PA = jnp.exp(-ALPHA * fg)
    Q = jnp.exp(-fgT)
    QA = jnp.exp(-ALPHA * fgT)

    head_outs = []
    for hi in range(NHEADS):
        h = h_all[:, hi * NHID:(hi + 1) * NHID]
        p = P[:, hi:hi + 1]
        pa = PA[:, hi:hi + 1]
        q = Q[NHEADS + hi:NHEADS + hi + 1, :]
        qa = QA[NHEADS + hi:NHEADS + hi + 1, :]
        head_outs.append(_elu(_att_prop(p, pa, q, qa, h, mask, NHID)))
    x1 = jnp.concatenate(head_outs, axis=1)

    h2 = jnp.dot(x1, wout_ref[...], preferred_element_type=jnp.float32)
    fg2 = jnp.dot(h2, a2_ref[...],
                  preferred_element_type=jnp.float32)
    fg2T = fg2.T
    f2 = fg2[:, 0:1]
    g2T = fg2T[1:2, :]
    out_ref[0] = _elu(_att_prop(
        jnp.exp(-f2), jnp.exp(-ALPHA * f2),
        jnp.exp(-g2T), jnp.exp(-ALPHA * g2T), h2, mask, NCLASS))


def kernel(x, adj, W_att, a_att, W_out, a_out):
    wc = W_att.transpose(1, 0, 2).reshape(NFEAT, NHEADS * NHID)
    a_src = a_att[:, 0, :NHID]
    a_dst = a_att[:, 0, NHID:]
    eye = jnp.eye(NHEADS, dtype=a_att.dtype)
    acomb = jnp.concatenate(
        [(a_src[:, :, None] * eye[:, None, :]).reshape(NHEADS * NHID, NHEADS),
         (a_dst[:, :, None] * eye[:, None, :]).reshape(NHEADS * NHID, NHEADS)],
        axis=1)
    a2 = jnp.concatenate(
        [a_out[0, :NCLASS, None], a_out[0, NCLASS:, None],
         jnp.zeros((NCLASS, 6), dtype=a_out.dtype)], axis=1)

    grid = (B,)
    return pl.pallas_call(
        _gat_body,
        grid=grid,
        in_specs=[
            pl.BlockSpec((1, N, NFEAT), lambda b: (b, 0, 0)),
            pl.BlockSpec((N, N), lambda b: (0, 0)),
            pl.BlockSpec((NFEAT, NHEADS * NHID), lambda b: (0, 0)),
            pl.BlockSpec((NHEADS * NHID, 2 * NHEADS), lambda b: (0, 0)),
            pl.BlockSpec((NHEADS * NHID, NCLASS), lambda b: (0, 0)),
            pl.BlockSpec((NCLASS, 8), lambda b: (0, 0)),
        ],
        out_specs=pl.BlockSpec((1, N, NCLASS), lambda b: (b, 0, 0)),
        out_shape=jax.ShapeDtypeStruct((B, N, NCLASS), jnp.float32),
    )(x, adj, wc, acomb, W_out, a2)


# speedup vs baseline: 2776.8989x; 1.2467x over previous
import jax
import jax.numpy as jnp
from jax.experimental import pallas as pl

NFEAT = 256
NHID = 32
NCLASS = 64
NHEADS = 8
ALPHA = 0.2
B = 4
N = 512

_NT = (((1,), (1,)), ((), ()))


def _elu(v):
    return jnp.where(v > 0, v, jnp.exp(jnp.minimum(v, 0.0)) - 1.0)


def _att_prop(f, gT, h, mask, d):
    p = jnp.exp(-f)
    pa = jnp.exp(-ALPHA * f)
    q = jnp.exp(-gT)
    qa = jnp.exp(-ALPHA * gT)
    e = jnp.minimum(p * q, pa * qa) * mask
    lane = jax.lax.broadcasted_iota(jnp.int32, (N, d), 1)
    ones_col = (lane == 0).astype(jnp.float32)
    rhs = jnp.concatenate([h, ones_col], axis=1)
    acc = jnp.dot(e.astype(jnp.bfloat16), rhs.astype(jnp.bfloat16),
                  preferred_element_type=jnp.float32)
    hp = acc[:, :d]
    rowsum = acc[:, d:d + 1]
    return hp * (1.0 / rowsum)


def _gat_body(x_ref, adj_ref, watt_ref, aatt_ref, wout_ref, aout_ref, out_ref):
    xb = x_ref[0]
    mask = (adj_ref[...] != 0).astype(jnp.float32)

    head_outs = []
    for hi in range(NHEADS):
        h = jnp.dot(xb, watt_ref[hi],
                    preferred_element_type=jnp.float32)
        a_src = aatt_ref[hi][:, :NHID]
        a_dst = aatt_ref[hi][:, NHID:]
        f = jax.lax.dot_general(h, a_src, _NT,
                                preferred_element_type=jnp.float32)
        gT = jax.lax.dot_general(a_dst, h, _NT,
                                 preferred_element_type=jnp.float32)
        head_outs.append(_elu(_att_prop(f, gT, h, mask, NHID)))
    x1 = jnp.concatenate(head_outs, axis=1)

    h2 = jnp.dot(x1, wout_ref[...], preferred_element_type=jnp.float32)
    a2_src = aout_ref[:, :NCLASS]
    a2_dst = aout_ref[:, NCLASS:]
    f2 = jax.lax.dot_general(h2, a2_src, _NT,
                             preferred_element_type=jnp.float32)
    g2T = jax.lax.dot_general(a2_dst, h2, _NT,
                              preferred_element_type=jnp.float32)
    out_ref[0] = _elu(_att_prop(f2, g2T, h2, mask, NCLASS))


def kernel(x, adj, W_att, a_att, W_out, a_out):
    grid = (B,)
    return pl.pallas_call(
        _gat_body,
        grid=grid,
        in_specs=[
            pl.BlockSpec((1, N, NFEAT), lambda b: (b, 0, 0)),
            pl.BlockSpec((N, N), lambda b: (0, 0)),
            pl.BlockSpec((NHEADS, NFEAT, NHID), lambda b: (0, 0, 0)),
            pl.BlockSpec((NHEADS, 1, 2 * NHID), lambda b: (0, 0, 0)),
            pl.BlockSpec((NHEADS * NHID, NCLASS), lambda b: (0, 0)),
            pl.BlockSpec((1, 2 * NCLASS), lambda b: (0, 0)),
        ],
        out_specs=pl.BlockSpec((1, N, NCLASS), lambda b: (b, 0, 0)),
        out_shape=jax.ShapeDtypeStruct((B, N, NCLASS), jnp.float32),
    )(x, adj, W_att, a_att, W_out, a_out)


# speedup vs baseline: 2842.7636x; 1.0237x over previous
import jax
import jax.numpy as jnp
from jax.experimental import pallas as pl

NFEAT = 256
NHID = 32
NCLASS = 64
NHEADS = 8
ALPHA = 0.2
B = 4
N = 512

_NT = (((1,), (1,)), ((), ()))


def _elu(v):
    return jnp.where(v > 0, v, jnp.exp(jnp.minimum(v, 0.0)) - 1.0)


def _att_prop(f, gT, h, mask, d):
    p = jnp.exp(-f).astype(jnp.bfloat16)
    pa = jnp.exp(-ALPHA * f).astype(jnp.bfloat16)
    q = jnp.exp(-gT).astype(jnp.bfloat16)
    qa = jnp.exp(-ALPHA * gT).astype(jnp.bfloat16)
    e = jnp.minimum(p * q, pa * qa) * mask
    lane = jax.lax.broadcasted_iota(jnp.int32, (N, d), 1)
    ones_col = (lane == 0).astype(jnp.float32)
    rhs = jnp.concatenate([h, ones_col], axis=1)
    acc = jnp.dot(e, rhs.astype(jnp.bfloat16),
                  preferred_element_type=jnp.float32)
    hp = acc[:, :d]
    rowsum = acc[:, d:d + 1]
    return hp * (1.0 / rowsum)


def _gat_body(x_ref, adj_ref, watt_ref, aatt_ref, wout_ref, aout_ref, out_ref):
    xb = x_ref[0]
    mask = (adj_ref[...] != 0).astype(jnp.bfloat16)

    head_outs = []
    for hi in range(NHEADS):
        h = jnp.dot(xb, watt_ref[hi],
                    preferred_element_type=jnp.float32)
        a_src = aatt_ref[hi][:, :NHID]
        a_dst = aatt_ref[hi][:, NHID:]
        f = jax.lax.dot_general(h, a_src, _NT,
                                preferred_element_type=jnp.float32)
        gT = jax.lax.dot_general(a_dst, h, _NT,
                                 preferred_element_type=jnp.float32)
        head_outs.append(_elu(_att_prop(f, gT, h, mask, NHID)))
    x1 = jnp.concatenate(head_outs, axis=1)

    h2 = jnp.dot(x1, wout_ref[...], preferred_element_type=jnp.float32)
    a2_src = aout_ref[:, :NCLASS]
    a2_dst = aout_ref[:, NCLASS:]
    f2 = jax.lax.dot_general(h2, a2_src, _NT,
                             preferred_element_type=jnp.float32)
    g2T = jax.lax.dot_general(a2_dst, h2, _NT,
                              preferred_element_type=jnp.float32)
    out_ref[0] = _elu(_att_prop(f2, g2T, h2, mask, NCLASS))


def kernel(x, adj, W_att, a_att, W_out, a_out):
    grid = (B,)
    return pl.pallas_call(
        _gat_body,
        grid=grid,
        in_specs=[
            pl.BlockSpec((1, N, NFEAT), lambda b: (b, 0, 0)),
            pl.BlockSpec((N, N), lambda b: (0, 0)),
            pl.BlockSpec((NHEADS, NFEAT, NHID), lambda b: (0, 0, 0)),
            pl.BlockSpec((NHEADS, 1, 2 * NHID), lambda b: (0, 0, 0)),
            pl.BlockSpec((NHEADS * NHID, NCLASS), lambda b: (0, 0)),
            pl.BlockSpec((1, 2 * NCLASS), lambda b: (0, 0)),
        ],
        out_specs=pl.BlockSpec((1, N, NCLASS), lambda b: (b, 0, 0)),
        out_shape=jax.ShapeDtypeStruct((B, N, NCLASS), jnp.float32),
    )(x, adj, W_att, a_att, W_out, a_out)


# speedup vs baseline: 3379.1310x; 1.1887x over previous
import jax
import jax.numpy as jnp
from jax.experimental import pallas as pl

NFEAT = 256
NHID = 32
NCLASS = 64
NHEADS = 8
ALPHA = 0.2
B = 4
N = 512

_NT = (((1,), (1,)), ((), ()))


def _elu(v):
    return jnp.where(v > 0, v, jnp.exp(jnp.minimum(v, 0.0)) - 1.0)


def _att_prop(p, pa, q, qa, h, mask, d):
    e = jnp.minimum(p * q, pa * qa) * mask
    lane = jax.lax.broadcasted_iota(jnp.int32, (N, d), 1)
    ones_col = (lane == 0).astype(jnp.float32)
    rhs = jnp.concatenate([h, ones_col], axis=1)
    acc = jnp.dot(e, rhs.astype(jnp.bfloat16),
                  preferred_element_type=jnp.float32)
    hp = acc[:, :d]
    rowsum = acc[:, d:d + 1]
    return hp * (1.0 / rowsum)


def _gat_body(x_ref, adj_ref, watt_ref, aatt_ref, wout_ref, aout_ref, out_ref):
    xb = x_ref[0]
    mask = (adj_ref[...] != 0).astype(jnp.bfloat16)

    wcat = jnp.concatenate([watt_ref[hi] for hi in range(NHEADS)],
                           axis=1)
    h_all = jnp.dot(xb, wcat, preferred_element_type=jnp.float32)

    aattT = aatt_ref[:, 0, :].T
    row_head = jax.lax.broadcasted_iota(
        jnp.int32, (NHEADS * NHID, NHEADS), 0) // NHID
    col_head = jax.lax.broadcasted_iota(
        jnp.int32, (NHEADS * NHID, NHEADS), 1)
    blk = (row_head == col_head).astype(jnp.float32)
    acomb = jnp.concatenate(
        [jnp.tile(aattT[:NHID, :], (NHEADS, 1)) * blk,
         jnp.tile(aattT[NHID:, :], (NHEADS, 1)) * blk],
        axis=1)
    fg = jnp.dot(h_all, acomb, preferred_element_type=jnp.float32)
    fgT = fg.T
    P = jnp.exp(-fg[:, :NHEADS]).astype(jnp.bfloat16)
    PA = jnp.exp(-ALPHA * fg[:, :NHEADS]).astype(jnp.bfloat16)
    Q = jnp.exp(-fgT[NHEADS:, :]).astype(jnp.bfloat16)
    QA = jnp.exp(-ALPHA * fgT[NHEADS:, :]).astype(jnp.bfloat16)

    head_outs = []
    for hi in range(NHEADS):
        h = h_all[:, hi * NHID:(hi + 1) * NHID]
        head_outs.append(_elu(_att_prop(
            P[:, hi:hi + 1], PA[:, hi:hi + 1],
            Q[hi:hi + 1, :], QA[hi:hi + 1, :], h, mask, NHID)))
    x1 = jnp.concatenate(head_outs, axis=1)

    h2 = jnp.dot(x1, wout_ref[...], preferred_element_type=jnp.float32)
    a2_src = aout_ref[:, :NCLASS]
    a2_dst = aout_ref[:, NCLASS:]
    f2 = jax.lax.dot_general(h2, a2_src, _NT,
                             preferred_element_type=jnp.float32)
    g2T = jax.lax.dot_general(a2_dst, h2, _NT,
                              preferred_element_type=jnp.float32)
    out_ref[0] = _elu(_att_prop(
        jnp.exp(-f2).astype(jnp.bfloat16),
        jnp.exp(-ALPHA * f2).astype(jnp.bfloat16),
        jnp.exp(-g2T).astype(jnp.bfloat16),
        jnp.exp(-ALPHA * g2T).astype(jnp.bfloat16), h2, mask, NCLASS))


def kernel(x, adj, W_att, a_att, W_out, a_out):
    grid = (B,)
    return pl.pallas_call(
        _gat_body,
        grid=grid,
        in_specs=[
            pl.BlockSpec((1, N, NFEAT), lambda b: (b, 0, 0)),
            pl.BlockSpec((N, N), lambda b: (0, 0)),
            pl.BlockSpec((NHEADS, NFEAT, NHID), lambda b: (0, 0, 0)),
            pl.BlockSpec((NHEADS, 1, 2 * NHID), lambda b: (0, 0, 0)),
            pl.BlockSpec((NHEADS * NHID, NCLASS), lambda b: (0, 0)),
            pl.BlockSpec((1, 2 * NCLASS), lambda b: (0, 0)),
        ],
        out_specs=pl.BlockSpec((1, N, NCLASS), lambda b: (b, 0, 0)),
        out_shape=jax.ShapeDtypeStruct((B, N, NCLASS), jnp.float32),
    )(x, adj, W_att, a_att, W_out, a_out)


# speedup vs baseline: 3672.6140x; 1.0869x over previous
import jax
import jax.numpy as jnp
from jax.experimental import pallas as pl

NFEAT = 256
NHID = 32
NCLASS = 64
NHEADS = 8
ALPHA = 0.2
B = 4
N = 512

_NT = (((1,), (1,)), ((), ()))


def _elu(v):
    return jnp.where(v > 0, v, jnp.exp(jnp.minimum(v, 0.0)) - 1.0)


def _att_prop(p, pa, q, qa, h, mask, d):
    e = jnp.minimum(p * q, pa * qa) * mask
    lane = jax.lax.broadcasted_iota(jnp.int32, (N, d), 1)
    ones_col = (lane == 0).astype(jnp.float32)
    rhs = jnp.concatenate([h, ones_col], axis=1)
    acc = jnp.dot(e, rhs.astype(jnp.bfloat16),
                  preferred_element_type=jnp.float32)
    hp = acc[:, :d]
    rowsum = acc[:, d:d + 1]
    return hp * (1.0 / rowsum)


def _gat_body(x_ref, adj_ref, watt_ref, aatt_ref, wout_ref, aout_ref, out_ref):
    mask = (adj_ref[...] != 0).astype(jnp.bfloat16)

    wcat = jnp.concatenate([watt_ref[hi] for hi in range(NHEADS)],
                           axis=1)

    aattT = aatt_ref[:, 0, :].T
    row_head = jax.lax.broadcasted_iota(
        jnp.int32, (NHEADS * NHID, NHEADS), 0) // NHID
    col_head = jax.lax.broadcasted_iota(
        jnp.int32, (NHEADS * NHID, NHEADS), 1)
    blk = (row_head == col_head).astype(jnp.float32)
    acomb = jnp.concatenate(
        [jnp.tile(aattT[:NHID, :], (NHEADS, 1)) * blk,
         jnp.tile(aattT[NHID:, :], (NHEADS, 1)) * blk],
        axis=1)

    a2_src = aout_ref[:, :NCLASS]
    a2_dst = aout_ref[:, NCLASS:]

    for b in range(B):
        xb = x_ref[b]
        h_all = jnp.dot(xb, wcat, preferred_element_type=jnp.float32)
        fg = jnp.dot(h_all, acomb,
                     preferred_element_type=jnp.float32)
        fgT = fg.T
        P = jnp.exp(-fg[:, :NHEADS]).astype(jnp.bfloat16)
        PA = jnp.exp(-ALPHA * fg[:, :NHEADS]).astype(jnp.bfloat16)
        Q = jnp.exp(-fgT[NHEADS:, :]).astype(jnp.bfloat16)
        QA = jnp.exp(-ALPHA * fgT[NHEADS:, :]).astype(jnp.bfloat16)

        head_outs = []
        for hi in range(NHEADS):
            h = h_all[:, hi * NHID:(hi + 1) * NHID]
            head_outs.append(_elu(_att_prop(
                P[:, hi:hi + 1], PA[:, hi:hi + 1],
                Q[hi:hi + 1, :], QA[hi:hi + 1, :], h, mask, NHID)))
        x1 = jnp.concatenate(head_outs, axis=1)

        h2 = jnp.dot(x1, wout_ref[...], preferred_element_type=jnp.float32)
        f2 = jax.lax.dot_general(h2, a2_src, _NT,
                                 preferred_element_type=jnp.float32)
        g2T = jax.lax.dot_general(a2_dst, h2, _NT,
                                  preferred_element_type=jnp.float32)
        out_ref[b] = _elu(_att_prop(
            jnp.exp(-f2).astype(jnp.bfloat16),
            jnp.exp(-ALPHA * f2).astype(jnp.bfloat16),
            jnp.exp(-g2T).astype(jnp.bfloat16),
            jnp.exp(-ALPHA * g2T).astype(jnp.bfloat16), h2, mask, NCLASS))


def kernel(x, adj, W_att, a_att, W_out, a_out):
    return pl.pallas_call(
        _gat_body,
        out_shape=jax.ShapeDtypeStruct((B, N, NCLASS), jnp.float32),
    )(x, adj, W_att, a_att, W_out, a_out)
